# baseline (speedup 1.0000x reference)
import functools

import jax
import jax.numpy as jnp
from jax import lax
from jax.experimental import pallas as pl
from jax.experimental.pallas import tpu as pltpu
from jax.experimental.pallas import tpu_sc as plsc

_D = 64
_NC, _NS = 2, 16
_NW = _NC * _NS
_V = 1000000
_FULL = _V // 128
_BPT = _FULL // _NW
_REM = _FULL - _BPT * _NW
_TAILN = _V - _FULL * 128

_N, _S = 4096, 200
_B = _N * _S
_JBLK = _B // 128
_JPT = _JBLK // _NW


def _mesh():
    return plsc.VectorSubcoreMesh(core_axis_name="c", subcore_axis_name="s")


def _iota16():
    return lax.iota(jnp.int32, 16)


def _splat(x):
    return jnp.full((16,), x, jnp.int32)


@jax.jit
def _detranspose(table_t):
    @functools.partial(
        pl.kernel,
        mesh=_mesh(),
        out_type=jax.ShapeDtypeStruct((_V * _D,), jnp.float32),
        scratch_types=[
            pltpu.VMEM((4, 8, 8, 128), jnp.float32),
            pltpu.VMEM((4 * 8192,), jnp.float32),
            pltpu.SemaphoreType.DMA,
            pltpu.SemaphoreType.DMA,
            pltpu.SemaphoreType.DMA,
            pltpu.SemaphoreType.DMA,
            pltpu.SemaphoreType.DMA,
            pltpu.SemaphoreType.DMA,
            pltpu.SemaphoreType.DMA,
            pltpu.SemaphoreType.DMA,
        ],
        compiler_params=pltpu.CompilerParams(use_tc_tiling_on_sc=True, needs_layout_passes=False),
    )
    def ka(tt_hbm, out_hbm, inb, outb,
           gi0, gi1, gi2, gi3, wo0, wo1, wo2, wo3):
        wid = lax.axis_index("s") * _NC + lax.axis_index("c")
        gsem = (gi0, gi1, gi2, gi3)
        wsem = (wo0, wo1, wo2, wo3)
        iot = _iota16()
        dtr = [(iot + d0) >> 3 for d0 in (0, 16, 32, 48)]
        ddr = [(iot + d0) & 7 for d0 in (0, 16, 32, 48)]

        def blk_of(i):
            return i * _NW + wid

        def load(i, b):
            blk = blk_of(i)
            return [
                pltpu.make_async_copy(
                    tt_hbm.at[pl.ds(tr * 8, 8), pl.ds(blk * 128, 128)],
                    inb.at[b, tr], gsem[b])
                for tr in range(8)
            ]

        def store(i, b):
            blk = blk_of(i)
            return pltpu.make_async_copy(
                outb.at[pl.ds(b * 8192, 8192)],
                out_hbm.at[pl.ds(blk * 8192, 8192)], wsem[b])

        def transpose(b):
            inb_b = inb.at[b]
            outb_b = outb.at[pl.ds(b * 8192, 8192)]

            def kbody(k, _):
                nmod = (iot + k) & 15
                srel = (nmod << 6) + iot
                for di in range(4):
                    d0 = 16 * di
                    for g in range(8):
                        n0 = 16 * g
                        nidx = nmod + n0
                        v = plsc.load_gather(inb_b, [dtr[di], ddr[di], nidx])
                        plsc.store_scatter(outb_b, [srel + (n0 * _D + d0)], v)
                return _
            lax.fori_loop(0, 16, kbody, 0)

        for b in range(4):
            for c in load(b, b):
                c.start()
            store(b, b).start()

        def body(q, carry):
            for b in range(4):
                i = 4 * q + b
                for c in load(i, b):
                    c.wait()
                store(i, b).wait()
                transpose(b)
                for c in load(jnp.minimum(i + 4, _BPT - 1), b):
                    c.start()
                store(i, b).start()
            return carry

        lax.fori_loop(0, _BPT // 4, body, 0)

        for b in range(4):
            store(0, b).wait()
            for c in load(0, b):
                c.wait()

        @pl.when(wid < _REM)
        def _():
            blk = _FULL - _REM + wid
            for tr in range(8):
                pltpu.sync_copy(
                    tt_hbm.at[pl.ds(tr * 8, 8), pl.ds(blk * 128, 128)],
                    inb.at[0, tr])
            transpose(0)
            pltpu.sync_copy(outb.at[pl.ds(0, 8192)],
                            out_hbm.at[pl.ds(blk * 8192, 8192)])


    return ka(table_t)


@jax.jit
def _gather_blocks(table_lin, idx_j, tail64):
    @functools.partial(
        pl.kernel,
        mesh=_mesh(),
        out_type=jax.ShapeDtypeStruct((_S, 8, _N // 128, 8, 128), jnp.float32),
        scratch_types=[
            pltpu.VMEM((_JPT * 128,), jnp.int32),
            pltpu.VMEM((_TAILN, _D), jnp.float32),
            pltpu.VMEM((4, 128, _D), jnp.float32),
            pltpu.VMEM((4, 8, 8, 128), jnp.float32),
            pltpu.SemaphoreType.DMA,
            pltpu.SemaphoreType.DMA,
            pltpu.SemaphoreType.DMA,
            pltpu.SemaphoreType.DMA,
            pltpu.SemaphoreType.DMA,
            pltpu.SemaphoreType.DMA,
            pltpu.SemaphoreType.DMA,
            pltpu.SemaphoreType.DMA,
        ],
        compiler_params=pltpu.CompilerParams(use_tc_tiling_on_sc=False, needs_layout_passes=False),
    )
    def kb(tab_hbm, idx_hbm, tail_hbm, out_hbm, idxv, tailv, rows, trans,
           g0, g1, g2, g3, w0, w1, w2, w3):
        wid = lax.axis_index("s") * _NC + lax.axis_index("c")
        gsem = (g0, g1, g2, g3)
        wsem = (w0, w1, w2, w3)
        iot = _iota16()
        nvec = [iot + 16 * g for g in range(8)]
        jb0 = wid * _JPT
        tail_lo = _FULL * 128

        pltpu.sync_copy(idx_hbm.at[pl.ds(jb0 * 128, _JPT * 128)], idxv)
        pltpu.sync_copy(tail_hbm, tailv)

        def fixup(i, b):
            accs = []
            masks = []
            tidxs = []
            for g in range(8):
                iv = idxv[pl.ds(i * 128 + 16 * g, 16)]
                m = iv >= tail_lo
                masks.append(m)
                tidxs.append(jnp.maximum(iv - tail_lo, 0))
                accs.append(m.astype(jnp.int32))
            acc = accs[0]
            for a in accs[1:]:
                acc = acc + a
            nhit = lax.reduce_max(acc, axes=(0,))

            @pl.when(nhit > 0)
            def _():
                def dbody(d4, _):
                    for u in range(4):
                        d = d4 * 4 + u
                        dsp = _splat(d)
                        for g in range(8):
                            v = plsc.load_gather(
                                tailv, [tidxs[g], dsp], mask=masks[g])
                            plsc.store_scatter(
                                rows.at[b], [nvec[g], dsp], v,
                                mask=masks[g])
                    return _
                lax.fori_loop(0, 16, dbody, 0)

        def gather(i, b):
            return pltpu.make_async_copy(
                tab_hbm.at[idxv.at[pl.ds(i * 128, 128)]], rows.at[b], gsem[b])

        def store(i, b):
            jb = jb0 + i
            s = jb // (_N // 128)
            tc = jb % (_N // 128)
            return pltpu.make_async_copy(
                trans.at[b], out_hbm.at[s, :, tc], wsem[b])

        def transpose(b):
            rows_b = rows.at[b]
            trans_b = trans.at[b]

            def kbody(k, _):
                dmod = (iot + k) & 15
                trrel = dmod >> 3
                rrel = dmod & 7
                for di in range(4):
                    d0 = 16 * di
                    didx = dmod + d0
                    trv = trrel + (d0 >> 3)
                    for g in range(8):
                        v = plsc.load_gather(rows_b, [nvec[g], didx])
                        plsc.store_scatter(trans_b, [trv, rrel, nvec[g]], v)
                return _
            lax.fori_loop(0, 16, kbody, 0)

        for b in range(4):
            gather(b, b).start()
            store(b, b).start()

        def body(q, carry):
            for b in range(4):
                i = 4 * q + b
                gather(i, b).wait()
                fixup(i, b)
                store(i, b).wait()
                transpose(b)
                gather(jnp.minimum(i + 4, _JPT - 1), b).start()
                store(i, b).start()
            return carry

        lax.fori_loop(0, _JPT // 4, body, 0)

        for b in range(4):
            store(0, b).wait()
            gather(0, b).wait()

    return kb(table_lin, idx_j, tail64)


def kernel(inputs, table):
    table_t = table.T
    tab_lin = _detranspose(table_t).reshape(_V, _D)
    idx_j = inputs.T.reshape(_B).astype(jnp.int32)
    tail64 = table[_FULL * 128:]
    out5 = _gather_blocks(tab_lin, idx_j, tail64)
    return out5.transpose(2, 4, 0, 1, 3).reshape(_N, _S, _D)


# speedup vs baseline: 1.0080x; 1.0080x over previous
import functools

import jax
import jax.numpy as jnp
from jax import lax
from jax.experimental import pallas as pl
from jax.experimental.pallas import tpu as pltpu
from jax.experimental.pallas import tpu_sc as plsc

_D = 64
_NC, _NS = 2, 16
_NW = _NC * _NS
_V = 1000000
_FULL = _V // 128
_BPT = _FULL // _NW
_REM = _FULL - _BPT * _NW
_TAILN = _V - _FULL * 128

_N, _S = 4096, 200
_B = _N * _S
_JBLK = _B // 128
_JPT = _JBLK // _NW


def _mesh():
    return plsc.VectorSubcoreMesh(core_axis_name="c", subcore_axis_name="s")


def _iota16():
    return lax.iota(jnp.int32, 16)


def _splat(x):
    return jnp.full((16,), x, jnp.int32)


@jax.jit
def _detranspose(table_t):
    @functools.partial(
        pl.kernel,
        mesh=_mesh(),
        out_type=jax.ShapeDtypeStruct((_V * _D,), jnp.float32),
        scratch_types=[
            pltpu.VMEM((256, 128), jnp.float32),
            pltpu.VMEM((4 * 8192,), jnp.float32),
            pltpu.SemaphoreType.DMA,
            pltpu.SemaphoreType.DMA,
            pltpu.SemaphoreType.DMA,
            pltpu.SemaphoreType.DMA,
            pltpu.SemaphoreType.DMA,
            pltpu.SemaphoreType.DMA,
            pltpu.SemaphoreType.DMA,
            pltpu.SemaphoreType.DMA,
        ],
        compiler_params=pltpu.CompilerParams(use_tc_tiling_on_sc=True, needs_layout_passes=False),
    )
    def ka(tt_hbm, out_hbm, inb, outb,
           gi0, gi1, gi2, gi3, wo0, wo1, wo2, wo3):
        wid = lax.axis_index("s") * _NC + lax.axis_index("c")
        gsem = (gi0, gi1, gi2, gi3)
        wsem = (wo0, wo1, wo2, wo3)
        iot = _iota16()

        def blk_of(i):
            return i * _NW + wid

        def load(i, b):
            blk = blk_of(i)
            return [
                pltpu.make_async_copy(
                    tt_hbm.at[:, pl.ds(blk * 128, 128)],
                    inb.at[pl.ds(b * 64, 64)], gsem[b])
            ]

        def store(i, b):
            blk = blk_of(i)
            return pltpu.make_async_copy(
                outb.at[pl.ds(b * 8192, 8192)],
                out_hbm.at[pl.ds(blk * 8192, 8192)], wsem[b])

        def transpose(b):
            inb_b = inb.at[pl.ds(b * 64, 64)]
            outb_b = outb.at[pl.ds(b * 8192, 8192)]

            def kbody(k, _):
                nmod = (iot + k) & 15
                srel = (nmod << 6) + iot
                for di in range(4):
                    d0 = 16 * di
                    for g in range(8):
                        n0 = 16 * g
                        nidx = nmod + n0
                        v = plsc.load_gather(inb_b, [iot + d0, nidx])
                        plsc.store_scatter(outb_b, [srel + (n0 * _D + d0)], v)
                return _
            lax.fori_loop(0, 16, kbody, 0)

        for b in range(4):
            for c in load(b, b):
                c.start()
            store(b, b).start()

        def body(q, carry):
            for b in range(4):
                i = 4 * q + b
                for c in load(i, b):
                    c.wait()
                store(i, b).wait()
                transpose(b)
                for c in load(jnp.minimum(i + 4, _BPT - 1), b):
                    c.start()
                store(i, b).start()
            return carry

        lax.fori_loop(0, _BPT // 4, body, 0)

        for b in range(4):
            store(0, b).wait()
            for c in load(0, b):
                c.wait()

        @pl.when(wid < _REM)
        def _():
            blk = _FULL - _REM + wid
            pltpu.sync_copy(
                tt_hbm.at[:, pl.ds(blk * 128, 128)],
                inb.at[pl.ds(0, 64)])
            transpose(0)
            pltpu.sync_copy(outb.at[pl.ds(0, 8192)],
                            out_hbm.at[pl.ds(blk * 8192, 8192)])


    return ka(table_t)


@jax.jit
def _gather_blocks(table_lin, idx_j, tail64):
    @functools.partial(
        pl.kernel,
        mesh=_mesh(),
        out_type=jax.ShapeDtypeStruct((_S, 8, _N // 128, 8, 128), jnp.float32),
        scratch_types=[
            pltpu.VMEM((_JPT * 128,), jnp.int32),
            pltpu.VMEM((_TAILN, _D), jnp.float32),
            pltpu.VMEM((4, 128, _D), jnp.float32),
            pltpu.VMEM((4, 8, 8, 128), jnp.float32),
            pltpu.SemaphoreType.DMA,
            pltpu.SemaphoreType.DMA,
            pltpu.SemaphoreType.DMA,
            pltpu.SemaphoreType.DMA,
            pltpu.SemaphoreType.DMA,
            pltpu.SemaphoreType.DMA,
            pltpu.SemaphoreType.DMA,
            pltpu.SemaphoreType.DMA,
        ],
        compiler_params=pltpu.CompilerParams(use_tc_tiling_on_sc=False, needs_layout_passes=False),
    )
    def kb(tab_hbm, idx_hbm, tail_hbm, out_hbm, idxv, tailv, rows, trans,
           g0, g1, g2, g3, w0, w1, w2, w3):
        wid = lax.axis_index("s") * _NC + lax.axis_index("c")
        gsem = (g0, g1, g2, g3)
        wsem = (w0, w1, w2, w3)
        iot = _iota16()
        nvec = [iot + 16 * g for g in range(8)]
        jb0 = wid * _JPT
        tail_lo = _FULL * 128

        pltpu.sync_copy(idx_hbm.at[pl.ds(jb0 * 128, _JPT * 128)], idxv)
        pltpu.sync_copy(tail_hbm, tailv)

        def fixup(i, b):
            accs = []
            masks = []
            tidxs = []
            for g in range(8):
                iv = idxv[pl.ds(i * 128 + 16 * g, 16)]
                m = iv >= tail_lo
                masks.append(m)
                tidxs.append(jnp.maximum(iv - tail_lo, 0))
                accs.append(m.astype(jnp.int32))
            acc = accs[0]
            for a in accs[1:]:
                acc = acc + a
            nhit = lax.reduce_max(acc, axes=(0,))

            @pl.when(nhit > 0)
            def _():
                def dbody(d4, _):
                    for u in range(4):
                        d = d4 * 4 + u
                        dsp = _splat(d)
                        for g in range(8):
                            v = plsc.load_gather(
                                tailv, [tidxs[g], dsp], mask=masks[g])
                            plsc.store_scatter(
                                rows.at[b], [nvec[g], dsp], v,
                                mask=masks[g])
                    return _
                lax.fori_loop(0, 16, dbody, 0)

        def gather(i, b):
            return pltpu.make_async_copy(
                tab_hbm.at[idxv.at[pl.ds(i * 128, 128)]], rows.at[b], gsem[b])

        def gather_wait(b):
            pltpu.make_async_copy(
                tab_hbm.at[pl.ds(0, 128)], rows.at[b], gsem[b]).wait()

        def store(i, b):
            jb = jb0 + i
            s = jb // (_N // 128)
            tc = jb % (_N // 128)
            return pltpu.make_async_copy(
                trans.at[b], out_hbm.at[s, :, tc], wsem[b])

        def transpose(b):
            rows_b = rows.at[b]
            trans_b = trans.at[b]

            def kbody(k, _):
                dmod = (iot + k) & 15
                trrel = dmod >> 3
                rrel = dmod & 7
                for di in range(4):
                    d0 = 16 * di
                    didx = dmod + d0
                    trv = trrel + (d0 >> 3)
                    for g in range(8):
                        v = plsc.load_gather(rows_b, [nvec[g], didx])
                        plsc.store_scatter(trans_b, [trv, rrel, nvec[g]], v)
                return _
            lax.fori_loop(0, 16, kbody, 0)

        for b in range(4):
            gather(b, b).start()
            store(b, b).start()

        def body(q, carry):
            for b in range(4):
                i = 4 * q + b
                gather_wait(b)
                fixup(i, b)
                store(i, b).wait()
                transpose(b)
                gather(jnp.minimum(i + 4, _JPT - 1), b).start()
                store(i, b).start()
            return carry

        lax.fori_loop(0, _JPT // 4, body, 0)

        for b in range(4):
            store(0, b).wait()
            gather_wait(b)

    return kb(table_lin, idx_j, tail64)


def kernel(inputs, table):
    table_t = table.T
    tab_lin = _detranspose(table_t).reshape(_V, _D)
    idx_j = inputs.T.reshape(_B).astype(jnp.int32)
    tail64 = table[_FULL * 128:]
    out5 = _gather_blocks(tab_lin, idx_j, tail64)
    return out5.transpose(2, 4, 0, 1, 3).reshape(_N, _S, _D)


# speedup vs baseline: 1.0092x; 1.0012x over previous
import functools

import jax
import jax.numpy as jnp
from jax import lax
from jax.experimental import pallas as pl
from jax.experimental.pallas import tpu as pltpu
from jax.experimental.pallas import tpu_sc as plsc

_D = 64
_NC, _NS = 2, 16
_NW = _NC * _NS
_V = 1000000
_FULL = _V // 128
_BPT = _FULL // _NW
_REM = _FULL - _BPT * _NW
_TAILN = _V - _FULL * 128

_N, _S = 4096, 200
_B = _N * _S
_JBLK = _B // 128
_JPT = _JBLK // _NW


def _mesh():
    return plsc.VectorSubcoreMesh(core_axis_name="c", subcore_axis_name="s")


def _iota16():
    return lax.iota(jnp.int32, 16)


def _splat(x):
    return jnp.full((16,), x, jnp.int32)


@jax.jit
def _detranspose(table_t):
    @functools.partial(
        pl.kernel,
        mesh=_mesh(),
        out_type=jax.ShapeDtypeStruct((_V * _D,), jnp.float32),
        scratch_types=[
            pltpu.VMEM((256, 128), jnp.float32),
            pltpu.VMEM((4 * 8192,), jnp.float32),
            pltpu.SemaphoreType.DMA,
            pltpu.SemaphoreType.DMA,
            pltpu.SemaphoreType.DMA,
            pltpu.SemaphoreType.DMA,
            pltpu.SemaphoreType.DMA,
            pltpu.SemaphoreType.DMA,
            pltpu.SemaphoreType.DMA,
            pltpu.SemaphoreType.DMA,
        ],
        compiler_params=pltpu.CompilerParams(use_tc_tiling_on_sc=True, needs_layout_passes=False, disable_bounds_checks=True),
    )
    def ka(tt_hbm, out_hbm, inb, outb,
           gi0, gi1, gi2, gi3, wo0, wo1, wo2, wo3):
        wid = lax.axis_index("s") * _NC + lax.axis_index("c")
        gsem = (gi0, gi1, gi2, gi3)
        wsem = (wo0, wo1, wo2, wo3)
        iot = _iota16()

        def blk_of(i):
            return i * _NW + wid

        def load(i, b):
            blk = blk_of(i)
            return [
                pltpu.make_async_copy(
                    tt_hbm.at[:, pl.ds(blk * 128, 128)],
                    inb.at[pl.ds(b * 64, 64)], gsem[b])
            ]

        def store(i, b):
            blk = blk_of(i)
            return pltpu.make_async_copy(
                outb.at[pl.ds(b * 8192, 8192)],
                out_hbm.at[pl.ds(blk * 8192, 8192)], wsem[b])

        def transpose(b):
            inb_b = inb.at[pl.ds(b * 64, 64)]
            outb_b = outb.at[pl.ds(b * 8192, 8192)]

            def kbody(k, _):
                nmod = (iot + k) & 15
                srel = (nmod << 6) + iot
                for di in range(4):
                    d0 = 16 * di
                    for g in range(8):
                        n0 = 16 * g
                        nidx = nmod + n0
                        v = plsc.load_gather(inb_b, [iot + d0, nidx])
                        plsc.store_scatter(outb_b, [srel + (n0 * _D + d0)], v)
                return _
            lax.fori_loop(0, 16, kbody, 0)

        for b in range(4):
            for c in load(b, b):
                c.start()
            store(b, b).start()

        def body(q, carry):
            for b in range(4):
                i = 4 * q + b
                for c in load(i, b):
                    c.wait()
                store(i, b).wait()
                transpose(b)
                for c in load(jnp.minimum(i + 4, _BPT - 1), b):
                    c.start()
                store(i, b).start()
            return carry

        lax.fori_loop(0, _BPT // 4, body, 0)

        for b in range(4):
            store(0, b).wait()
            for c in load(0, b):
                c.wait()

        @pl.when(wid < _REM)
        def _():
            blk = _FULL - _REM + wid
            pltpu.sync_copy(
                tt_hbm.at[:, pl.ds(blk * 128, 128)],
                inb.at[pl.ds(0, 64)])
            transpose(0)
            pltpu.sync_copy(outb.at[pl.ds(0, 8192)],
                            out_hbm.at[pl.ds(blk * 8192, 8192)])


    return ka(table_t)


@jax.jit
def _gather_blocks(table_lin, idx_j, tail64):
    @functools.partial(
        pl.kernel,
        mesh=_mesh(),
        out_type=jax.ShapeDtypeStruct((_S, 8, _N // 128, 8, 128), jnp.float32),
        scratch_types=[
            pltpu.VMEM((_JPT * 128,), jnp.int32),
            pltpu.VMEM((_TAILN, _D), jnp.float32),
            pltpu.VMEM((4, 128, _D), jnp.float32),
            pltpu.VMEM((4, 8, 8, 128), jnp.float32),
            pltpu.SemaphoreType.DMA,
            pltpu.SemaphoreType.DMA,
            pltpu.SemaphoreType.DMA,
            pltpu.SemaphoreType.DMA,
            pltpu.SemaphoreType.DMA,
            pltpu.SemaphoreType.DMA,
            pltpu.SemaphoreType.DMA,
            pltpu.SemaphoreType.DMA,
        ],
        compiler_params=pltpu.CompilerParams(use_tc_tiling_on_sc=False, needs_layout_passes=False, disable_bounds_checks=True),
    )
    def kb(tab_hbm, idx_hbm, tail_hbm, out_hbm, idxv, tailv, rows, trans,
           g0, g1, g2, g3, w0, w1, w2, w3):
        wid = lax.axis_index("s") * _NC + lax.axis_index("c")
        gsem = (g0, g1, g2, g3)
        wsem = (w0, w1, w2, w3)
        iot = _iota16()
        nvec = [iot + 16 * g for g in range(8)]
        jb0 = wid * _JPT
        tail_lo = _FULL * 128

        pltpu.sync_copy(idx_hbm.at[pl.ds(jb0 * 128, _JPT * 128)], idxv)
        pltpu.sync_copy(tail_hbm, tailv)

        def fixup(i, b):
            accs = []
            masks = []
            tidxs = []
            for g in range(8):
                iv = idxv[pl.ds(i * 128 + 16 * g, 16)]
                m = iv >= tail_lo
                masks.append(m)
                tidxs.append(jnp.maximum(iv - tail_lo, 0))
                accs.append(m.astype(jnp.int32))
            acc = accs[0]
            for a in accs[1:]:
                acc = acc + a
            nhit = lax.reduce_max(acc, axes=(0,))

            @pl.when(nhit > 0)
            def _():
                def dbody(d4, _):
                    for u in range(4):
                        d = d4 * 4 + u
                        dsp = _splat(d)
                        for g in range(8):
                            v = plsc.load_gather(
                                tailv, [tidxs[g], dsp], mask=masks[g])
                            plsc.store_scatter(
                                rows.at[b], [nvec[g], dsp], v,
                                mask=masks[g])
                    return _
                lax.fori_loop(0, 16, dbody, 0)

        def gather(i, b):
            return pltpu.make_async_copy(
                tab_hbm.at[idxv.at[pl.ds(i * 128, 128)]], rows.at[b], gsem[b])

        def gather_wait(b):
            pltpu.make_async_copy(
                tab_hbm.at[pl.ds(0, 128)], rows.at[b], gsem[b]).wait()

        def store(i, b):
            jb = jb0 + i
            s = jb // (_N // 128)
            tc = jb % (_N // 128)
            return pltpu.make_async_copy(
                trans.at[b], out_hbm.at[s, :, tc], wsem[b])

        def transpose(b):
            rows_b = rows.at[b]
            trans_b = trans.at[b]

            def kbody(k, _):
                dmod = (iot + k) & 15
                trrel = dmod >> 3
                rrel = dmod & 7
                for di in range(4):
                    d0 = 16 * di
                    didx = dmod + d0
                    trv = trrel + (d0 >> 3)
                    for g in range(8):
                        v = plsc.load_gather(rows_b, [nvec[g], didx])
                        plsc.store_scatter(trans_b, [trv, rrel, nvec[g]], v)
                return _
            lax.fori_loop(0, 16, kbody, 0)

        for b in range(4):
            gather(b, b).start()
            store(b, b).start()

        def body(q, carry):
            for b in range(4):
                i = 4 * q + b
                gather_wait(b)
                fixup(i, b)
                store(i, b).wait()
                transpose(b)
                gather(jnp.minimum(i + 4, _JPT - 1), b).start()
                store(i, b).start()
            return carry

        lax.fori_loop(0, _JPT // 4, body, 0)

        for b in range(4):
            store(0, b).wait()
            gather_wait(b)

    return kb(table_lin, idx_j, tail64)


def kernel(inputs, table):
    table_t = table.T
    tab_lin = _detranspose(table_t).reshape(_V, _D)
    idx_j = inputs.T.reshape(_B).astype(jnp.int32)
    tail64 = table[_FULL * 128:]
    out5 = _gather_blocks(tab_lin, idx_j, tail64)
    return out5.transpose(2, 4, 0, 1, 3).reshape(_N, _S, _D)


# speedup vs baseline: 1.6823x; 1.6670x over previous
import functools

import jax
import jax.numpy as jnp
from jax import lax
from jax.experimental import pallas as pl
from jax.experimental.pallas import tpu as pltpu
from jax.experimental.pallas import tpu_sc as plsc

_D = 64
_NC, _NS = 2, 16
_NW = _NC * _NS
_V = 1000000
_FULL = _V // 128
_BPT = _FULL // _NW
_REM = _FULL - _BPT * _NW
_TAILN = _V - _FULL * 128

_N, _S = 4096, 200
_B = _N * _S
_JBLK = _B // 128
_JPT = _JBLK // _NW


def _mesh():
    return plsc.VectorSubcoreMesh(core_axis_name="c", subcore_axis_name="s")


def _iota16():
    return lax.iota(jnp.int32, 16)


def _splat(x):
    return jnp.full((16,), x, jnp.int32)


@jax.jit
def _detranspose(table_t):
    @functools.partial(
        pl.kernel,
        mesh=_mesh(),
        out_type=jax.ShapeDtypeStruct((_V * _D,), jnp.float32),
        scratch_types=[
            pltpu.VMEM((256, 128), jnp.float32),
            pltpu.VMEM((4 * 8192,), jnp.float32),
            pltpu.SemaphoreType.DMA,
            pltpu.SemaphoreType.DMA,
            pltpu.SemaphoreType.DMA,
            pltpu.SemaphoreType.DMA,
            pltpu.SemaphoreType.DMA,
            pltpu.SemaphoreType.DMA,
            pltpu.SemaphoreType.DMA,
            pltpu.SemaphoreType.DMA,
        ],
        compiler_params=pltpu.CompilerParams(use_tc_tiling_on_sc=True, needs_layout_passes=False, disable_bounds_checks=True),
    )
    def ka(tt_hbm, out_hbm, inb, outb,
           gi0, gi1, gi2, gi3, wo0, wo1, wo2, wo3):
        wid = lax.axis_index("s") * _NC + lax.axis_index("c")
        gsem = (gi0, gi1, gi2, gi3)
        wsem = (wo0, wo1, wo2, wo3)
        iot = _iota16()

        def blk_of(i):
            return i * _NW + wid

        def load(i, b):
            blk = blk_of(i)
            return [
                pltpu.make_async_copy(
                    tt_hbm.at[:, pl.ds(blk * 128, 128)],
                    inb.at[pl.ds(b * 64, 64)], gsem[b])
            ]

        def store(i, b):
            blk = blk_of(i)
            return pltpu.make_async_copy(
                outb.at[pl.ds(b * 8192, 8192)],
                out_hbm.at[pl.ds(blk * 8192, 8192)], wsem[b])

        def transpose(b):
            inb_b = inb.at[pl.ds(b * 64, 64)]
            outb_b = outb.at[pl.ds(b * 8192, 8192)]

            def kbody(k, _):
                nmod = (iot + k) & 15
                srel = (nmod << 6) + iot
                vs = []
                for di in range(4):
                    d0 = 16 * di
                    for g in range(8):
                        n0 = 16 * g
                        vs.append(plsc.load_gather(
                            inb_b, [iot + d0, nmod + n0]))
                u = 0
                for di in range(4):
                    d0 = 16 * di
                    for g in range(8):
                        n0 = 16 * g
                        plsc.store_scatter(
                            outb_b, [srel + (n0 * _D + d0)], vs[u])
                        u += 1
                return _
            lax.fori_loop(0, 16, kbody, 0)

        for b in range(4):
            for c in load(b, b):
                c.start()
            store(b, b).start()

        def body(q, carry):
            for b in range(4):
                i = 4 * q + b
                for c in load(i, b):
                    c.wait()
                store(i, b).wait()
                transpose(b)
                for c in load(jnp.minimum(i + 4, _BPT - 1), b):
                    c.start()
                store(i, b).start()
            return carry

        lax.fori_loop(0, _BPT // 4, body, 0)

        for b in range(4):
            store(0, b).wait()
            for c in load(0, b):
                c.wait()

        @pl.when(wid < _REM)
        def _():
            blk = _FULL - _REM + wid
            pltpu.sync_copy(
                tt_hbm.at[:, pl.ds(blk * 128, 128)],
                inb.at[pl.ds(0, 64)])
            transpose(0)
            pltpu.sync_copy(outb.at[pl.ds(0, 8192)],
                            out_hbm.at[pl.ds(blk * 8192, 8192)])


    return ka(table_t)


@jax.jit
def _gather_blocks(table_lin, idx_j, tail64):
    @functools.partial(
        pl.kernel,
        mesh=_mesh(),
        out_type=jax.ShapeDtypeStruct((_S, 8, _N // 128, 8, 128), jnp.float32),
        scratch_types=[
            pltpu.VMEM((_JPT * 128,), jnp.int32),
            pltpu.VMEM((_TAILN, _D), jnp.float32),
            pltpu.VMEM((4, 128, _D), jnp.float32),
            pltpu.VMEM((4, 8, 8, 128), jnp.float32),
            pltpu.SemaphoreType.DMA,
            pltpu.SemaphoreType.DMA,
            pltpu.SemaphoreType.DMA,
            pltpu.SemaphoreType.DMA,
            pltpu.SemaphoreType.DMA,
            pltpu.SemaphoreType.DMA,
            pltpu.SemaphoreType.DMA,
            pltpu.SemaphoreType.DMA,
        ],
        compiler_params=pltpu.CompilerParams(use_tc_tiling_on_sc=False, needs_layout_passes=False, disable_bounds_checks=True),
    )
    def kb(tab_hbm, idx_hbm, tail_hbm, out_hbm, idxv, tailv, rows, trans,
           g0, g1, g2, g3, w0, w1, w2, w3):
        wid = lax.axis_index("s") * _NC + lax.axis_index("c")
        gsem = (g0, g1, g2, g3)
        wsem = (w0, w1, w2, w3)
        iot = _iota16()
        nvec = [iot + 16 * g for g in range(8)]
        jb0 = wid * _JPT
        tail_lo = _FULL * 128

        pltpu.sync_copy(idx_hbm.at[pl.ds(jb0 * 128, _JPT * 128)], idxv)
        pltpu.sync_copy(tail_hbm, tailv)

        def fixup(i, b):
            accs = []
            masks = []
            tidxs = []
            for g in range(8):
                iv = idxv[pl.ds(i * 128 + 16 * g, 16)]
                m = iv >= tail_lo
                masks.append(m)
                tidxs.append(jnp.maximum(iv - tail_lo, 0))
                accs.append(m.astype(jnp.int32))
            acc = accs[0]
            for a in accs[1:]:
                acc = acc + a
            nhit = lax.reduce_max(acc, axes=(0,))

            @pl.when(nhit > 0)
            def _():
                def dbody(d4, _):
                    for u in range(4):
                        d = d4 * 4 + u
                        dsp = _splat(d)
                        for g in range(8):
                            v = plsc.load_gather(
                                tailv, [tidxs[g], dsp], mask=masks[g])
                            plsc.store_scatter(
                                rows.at[b], [nvec[g], dsp], v,
                                mask=masks[g])
                    return _
                lax.fori_loop(0, 16, dbody, 0)

        def gather(i, b):
            return pltpu.make_async_copy(
                tab_hbm.at[idxv.at[pl.ds(i * 128, 128)]], rows.at[b], gsem[b])

        def gather_wait(b):
            pltpu.make_async_copy(
                tab_hbm.at[pl.ds(0, 128)], rows.at[b], gsem[b]).wait()

        def store(i, b):
            jb = jb0 + i
            s = jb // (_N // 128)
            tc = jb % (_N // 128)
            return pltpu.make_async_copy(
                trans.at[b], out_hbm.at[s, :, tc], wsem[b])

        def transpose(b):
            rows_b = rows.at[b]
            trans_b = trans.at[b]

            def kbody(k, _):
                dmod = (iot + k) & 15
                trrel = dmod >> 3
                rrel = dmod & 7
                vs = []
                for di in range(4):
                    didx = dmod + 16 * di
                    for g in range(8):
                        vs.append(plsc.load_gather(rows_b, [nvec[g], didx]))
                u = 0
                for di in range(4):
                    trv = trrel + 2 * di
                    for g in range(8):
                        plsc.store_scatter(
                            trans_b, [trv, rrel, nvec[g]], vs[u])
                        u += 1
                return _
            lax.fori_loop(0, 16, kbody, 0)

        for b in range(4):
            gather(b, b).start()
            store(b, b).start()

        def body(q, carry):
            for b in range(4):
                i = 4 * q + b
                gather_wait(b)
                fixup(i, b)
                store(i, b).wait()
                transpose(b)
                gather(jnp.minimum(i + 4, _JPT - 1), b).start()
                store(i, b).start()
            return carry

        lax.fori_loop(0, _JPT // 4, body, 0)

        for b in range(4):
            store(0, b).wait()
            gather_wait(b)

    return kb(table_lin, idx_j, tail64)


def kernel(inputs, table):
    table_t = table.T
    tab_lin = _detranspose(table_t).reshape(_V, _D)
    idx_j = inputs.T.reshape(_B).astype(jnp.int32)
    tail64 = table[_FULL * 128:]
    out5 = _gather_blocks(tab_lin, idx_j, tail64)
    return out5.transpose(2, 4, 0, 1, 3).reshape(_N, _S, _D)


# speedup vs baseline: 1.8297x; 1.0876x over previous
import functools

import jax
import jax.numpy as jnp
from jax import lax
from jax.experimental import pallas as pl
from jax.experimental.pallas import tpu as pltpu
from jax.experimental.pallas import tpu_sc as plsc

_D = 64
_NC, _NS = 2, 16
_NW = _NC * _NS
_V = 1000000
_FULL = _V // 128
_BPT = _FULL // _NW
_REM = _FULL - _BPT * _NW
_TAILN = _V - _FULL * 128

_N, _S = 4096, 200
_B = _N * _S
_JBLK = _B // 128
_JPT = _JBLK // _NW


def _mesh():
    return plsc.VectorSubcoreMesh(core_axis_name="c", subcore_axis_name="s")


def _iota16():
    return lax.iota(jnp.int32, 16)


def _splat(x):
    return jnp.full((16,), x, jnp.int32)


@jax.jit
def _detranspose(table_t):
    @functools.partial(
        pl.kernel,
        mesh=_mesh(),
        out_type=jax.ShapeDtypeStruct((_V * _D,), jnp.float32),
        scratch_types=[
            pltpu.VMEM((256, 128), jnp.float32),
            pltpu.VMEM((4 * 8192,), jnp.float32),
            pltpu.SemaphoreType.DMA,
            pltpu.SemaphoreType.DMA,
            pltpu.SemaphoreType.DMA,
            pltpu.SemaphoreType.DMA,
            pltpu.SemaphoreType.DMA,
            pltpu.SemaphoreType.DMA,
            pltpu.SemaphoreType.DMA,
            pltpu.SemaphoreType.DMA,
        ],
        compiler_params=pltpu.CompilerParams(use_tc_tiling_on_sc=True, needs_layout_passes=False, disable_bounds_checks=True),
    )
    def ka(tt_hbm, out_hbm, inb, outb,
           gi0, gi1, gi2, gi3, wo0, wo1, wo2, wo3):
        wid = lax.axis_index("s") * _NC + lax.axis_index("c")
        gsem = (gi0, gi1, gi2, gi3)
        wsem = (wo0, wo1, wo2, wo3)
        iot = _iota16()

        def blk_of(i):
            return i * _NW + wid

        def load(i, b):
            blk = blk_of(i)
            return [
                pltpu.make_async_copy(
                    tt_hbm.at[:, pl.ds(blk * 128, 128)],
                    inb.at[pl.ds(b * 64, 64)], gsem[b])
            ]

        def store(i, b):
            blk = blk_of(i)
            return pltpu.make_async_copy(
                outb.at[pl.ds(b * 8192, 8192)],
                out_hbm.at[pl.ds(blk * 8192, 8192)], wsem[b])

        def transpose(b):
            inb_b = inb.at[pl.ds(b * 64, 64)]
            outb_b = outb.at[pl.ds(b * 8192, 8192)]

            def kbody(k, _):
                nmod = (iot + k) & 15
                srel = (nmod << 6) + iot
                for half in range(2):
                    vs = []
                    for di in range(2 * half, 2 * half + 2):
                        d0 = 16 * di
                        for g in range(8):
                            n0 = 16 * g
                            vs.append(plsc.load_gather(
                                inb_b, [iot + d0, nmod + n0]))
                    u = 0
                    for di in range(2 * half, 2 * half + 2):
                        d0 = 16 * di
                        for g in range(8):
                            n0 = 16 * g
                            plsc.store_scatter(
                                outb_b, [srel + (n0 * _D + d0)], vs[u])
                            u += 1
                return _
            lax.fori_loop(0, 16, kbody, 0)

        for b in range(4):
            for c in load(b, b):
                c.start()
            store(b, b).start()

        def body(q, carry):
            for b in range(4):
                i = 4 * q + b
                for c in load(i, b):
                    c.wait()
                store(i, b).wait()
                transpose(b)
                for c in load(jnp.minimum(i + 4, _BPT - 1), b):
                    c.start()
                store(i, b).start()
            return carry

        lax.fori_loop(0, _BPT // 4, body, 0)

        for b in range(4):
            store(0, b).wait()
            for c in load(0, b):
                c.wait()

        @pl.when(wid < _REM)
        def _():
            blk = _FULL - _REM + wid
            pltpu.sync_copy(
                tt_hbm.at[:, pl.ds(blk * 128, 128)],
                inb.at[pl.ds(0, 64)])
            transpose(0)
            pltpu.sync_copy(outb.at[pl.ds(0, 8192)],
                            out_hbm.at[pl.ds(blk * 8192, 8192)])


    return ka(table_t)


@jax.jit
def _gather_blocks(table_lin, idx_j, tail64):
    @functools.partial(
        pl.kernel,
        mesh=_mesh(),
        out_type=jax.ShapeDtypeStruct((_S, 8, _N // 128, 8, 128), jnp.float32),
        scratch_types=[
            pltpu.VMEM((_JPT * 128,), jnp.int32),
            pltpu.VMEM((_TAILN, _D), jnp.float32),
            pltpu.VMEM((4, 128, _D), jnp.float32),
            pltpu.VMEM((4, 8, 8, 128), jnp.float32),
            pltpu.SemaphoreType.DMA,
            pltpu.SemaphoreType.DMA,
            pltpu.SemaphoreType.DMA,
            pltpu.SemaphoreType.DMA,
            pltpu.SemaphoreType.DMA,
            pltpu.SemaphoreType.DMA,
            pltpu.SemaphoreType.DMA,
            pltpu.SemaphoreType.DMA,
        ],
        compiler_params=pltpu.CompilerParams(use_tc_tiling_on_sc=False, needs_layout_passes=False, disable_bounds_checks=True),
    )
    def kb(tab_hbm, idx_hbm, tail_hbm, out_hbm, idxv, tailv, rows, trans,
           g0, g1, g2, g3, w0, w1, w2, w3):
        wid = lax.axis_index("s") * _NC + lax.axis_index("c")
        gsem = (g0, g1, g2, g3)
        wsem = (w0, w1, w2, w3)
        iot = _iota16()
        nvec = [iot + 16 * g for g in range(8)]
        jb0 = wid * _JPT
        tail_lo = _FULL * 128

        pltpu.sync_copy(idx_hbm.at[pl.ds(jb0 * 128, _JPT * 128)], idxv)
        pltpu.sync_copy(tail_hbm, tailv)

        def fixup(i, b):
            accs = []
            masks = []
            tidxs = []
            for g in range(8):
                iv = idxv[pl.ds(i * 128 + 16 * g, 16)]
                m = iv >= tail_lo
                masks.append(m)
                tidxs.append(jnp.maximum(iv - tail_lo, 0))
                accs.append(m.astype(jnp.int32))
            acc = accs[0]
            for a in accs[1:]:
                acc = acc + a
            nhit = lax.reduce_max(acc, axes=(0,))

            @pl.when(nhit > 0)
            def _():
                def dbody(d4, _):
                    for u in range(4):
                        d = d4 * 4 + u
                        dsp = _splat(d)
                        for g in range(8):
                            v = plsc.load_gather(
                                tailv, [tidxs[g], dsp], mask=masks[g])
                            plsc.store_scatter(
                                rows.at[b], [nvec[g], dsp], v,
                                mask=masks[g])
                    return _
                lax.fori_loop(0, 16, dbody, 0)

        def gather(i, b):
            return pltpu.make_async_copy(
                tab_hbm.at[idxv.at[pl.ds(i * 128, 128)]], rows.at[b], gsem[b])

        def gather_wait(b):
            pltpu.make_async_copy(
                tab_hbm.at[pl.ds(0, 128)], rows.at[b], gsem[b]).wait()

        def store(i, b):
            jb = jb0 + i
            s = jb // (_N // 128)
            tc = jb % (_N // 128)
            return pltpu.make_async_copy(
                trans.at[b], out_hbm.at[s, :, tc], wsem[b])

        def transpose(b):
            rows_b = rows.at[b]
            trans_b = trans.at[b]

            def kbody(k, _):
                dmod = (iot + k) & 15
                trrel = dmod >> 3
                rrel = dmod & 7
                for half in range(2):
                    vs = []
                    for di in range(2 * half, 2 * half + 2):
                        didx = dmod + 16 * di
                        for g in range(8):
                            vs.append(plsc.load_gather(
                                rows_b, [nvec[g], didx]))
                    u = 0
                    for di in range(2 * half, 2 * half + 2):
                        trv = trrel + 2 * di
                        for g in range(8):
                            plsc.store_scatter(
                                trans_b, [trv, rrel, nvec[g]], vs[u])
                            u += 1
                return _
            lax.fori_loop(0, 16, kbody, 0)

        for b in range(4):
            gather(b, b).start()
            store(b, b).start()

        def body(q, carry):
            for b in range(4):
                i = 4 * q + b
                gather_wait(b)
                fixup(i, b)
                store(i, b).wait()
                transpose(b)
                gather(jnp.minimum(i + 4, _JPT - 1), b).start()
                store(i, b).start()
            return carry

        lax.fori_loop(0, _JPT // 4, body, 0)

        for b in range(4):
            store(0, b).wait()
            gather_wait(b)

    return kb(table_lin, idx_j, tail64)


def kernel(inputs, table):
    table_t = table.T
    tab_lin = _detranspose(table_t).reshape(_V, _D)
    idx_j = inputs.T.reshape(_B).astype(jnp.int32)
    tail64 = table[_FULL * 128:]
    out5 = _gather_blocks(tab_lin, idx_j, tail64)
    return out5.transpose(2, 4, 0, 1, 3).reshape(_N, _S, _D)


# speedup vs baseline: 2.2287x; 1.2180x over previous
import functools

import jax
import jax.numpy as jnp
from jax import lax
from jax.experimental import pallas as pl
from jax.experimental.pallas import tpu as pltpu
from jax.experimental.pallas import tpu_sc as plsc

_D = 64
_NC, _NS = 2, 16
_NW = _NC * _NS
_V = 1000000
_FULL = _V // 128
_BPT = _FULL // _NW
_REM = _FULL - _BPT * _NW
_TAILN = _V - _FULL * 128

_N, _S = 4096, 200
_B = _N * _S
_JBLK = _B // 128
_JPT = _JBLK // _NW


def _mesh():
    return plsc.VectorSubcoreMesh(core_axis_name="c", subcore_axis_name="s")


def _iota16():
    return lax.iota(jnp.int32, 16)


def _splat(x):
    return jnp.full((16,), x, jnp.int32)


@jax.jit
def _detranspose(table_t):
    @functools.partial(
        pl.kernel,
        mesh=_mesh(),
        out_type=jax.ShapeDtypeStruct((_V * _D,), jnp.float32),
        scratch_types=[
            pltpu.VMEM((256, 128), jnp.float32),
            pltpu.VMEM((4 * 8192,), jnp.float32),
            pltpu.SemaphoreType.DMA,
            pltpu.SemaphoreType.DMA,
            pltpu.SemaphoreType.DMA,
            pltpu.SemaphoreType.DMA,
            pltpu.SemaphoreType.DMA,
            pltpu.SemaphoreType.DMA,
            pltpu.SemaphoreType.DMA,
            pltpu.SemaphoreType.DMA,
        ],
        compiler_params=pltpu.CompilerParams(use_tc_tiling_on_sc=True, needs_layout_passes=False, disable_bounds_checks=True),
    )
    def ka(tt_hbm, out_hbm, inb, outb,
           gi0, gi1, gi2, gi3, wo0, wo1, wo2, wo3):
        wid = lax.axis_index("s") * _NC + lax.axis_index("c")
        gsem = (gi0, gi1, gi2, gi3)
        wsem = (wo0, wo1, wo2, wo3)
        iot = _iota16()

        def blk_of(i):
            return i * _NW + wid

        def load(i, b):
            blk = blk_of(i)
            return [
                pltpu.make_async_copy(
                    tt_hbm.at[:, pl.ds(blk * 128, 128)],
                    inb.at[pl.ds(b * 64, 64)], gsem[b])
            ]

        def store(i, b):
            blk = blk_of(i)
            return pltpu.make_async_copy(
                outb.at[pl.ds(b * 8192, 8192)],
                out_hbm.at[pl.ds(blk * 8192, 8192)], wsem[b])

        def transpose(b):
            inb_b = inb.at[pl.ds(b * 64, 64)]
            outb_b = outb.at[pl.ds(b * 8192, 8192)]

            def kbody(k, _):
                nmod = (iot + k) & 15
                srel = (nmod << 6) + iot
                for quar in range(4):
                    d0 = 16 * quar
                    vs = []
                    for g in range(8):
                        n0 = 16 * g
                        vs.append(plsc.load_gather(
                            inb_b, [iot + d0, nmod + n0]))
                    for g in range(8):
                        n0 = 16 * g
                        plsc.store_scatter(
                            outb_b, [srel + (n0 * _D + d0)], vs[g])
                return _
            lax.fori_loop(0, 16, kbody, 0)

        for b in range(4):
            for c in load(b, b):
                c.start()
            store(b, b).start()

        def body(q, carry):
            for b in range(4):
                i = 4 * q + b
                for c in load(i, b):
                    c.wait()
                store(i, b).wait()
                transpose(b)
                for c in load(jnp.minimum(i + 4, _BPT - 1), b):
                    c.start()
                store(i, b).start()
            return carry

        lax.fori_loop(0, _BPT // 4, body, 0)

        for b in range(4):
            store(0, b).wait()
            for c in load(0, b):
                c.wait()

        @pl.when(wid < _REM)
        def _():
            blk = _FULL - _REM + wid
            pltpu.sync_copy(
                tt_hbm.at[:, pl.ds(blk * 128, 128)],
                inb.at[pl.ds(0, 64)])
            transpose(0)
            pltpu.sync_copy(outb.at[pl.ds(0, 8192)],
                            out_hbm.at[pl.ds(blk * 8192, 8192)])


    return ka(table_t)


@jax.jit
def _gather_blocks(table_lin, idx_j, tail64):
    @functools.partial(
        pl.kernel,
        mesh=_mesh(),
        out_type=jax.ShapeDtypeStruct((_S, 8, _N // 128, 8, 128), jnp.float32),
        scratch_types=[
            pltpu.VMEM((_JPT * 128,), jnp.int32),
            pltpu.VMEM((_TAILN, _D), jnp.float32),
            pltpu.VMEM((4, 128, _D), jnp.float32),
            pltpu.VMEM((4, 8, 8, 128), jnp.float32),
            pltpu.SemaphoreType.DMA,
            pltpu.SemaphoreType.DMA,
            pltpu.SemaphoreType.DMA,
            pltpu.SemaphoreType.DMA,
            pltpu.SemaphoreType.DMA,
            pltpu.SemaphoreType.DMA,
            pltpu.SemaphoreType.DMA,
            pltpu.SemaphoreType.DMA,
        ],
        compiler_params=pltpu.CompilerParams(use_tc_tiling_on_sc=False, needs_layout_passes=False, disable_bounds_checks=True),
    )
    def kb(tab_hbm, idx_hbm, tail_hbm, out_hbm, idxv, tailv, rows, trans,
           g0, g1, g2, g3, w0, w1, w2, w3):
        wid = lax.axis_index("s") * _NC + lax.axis_index("c")
        gsem = (g0, g1, g2, g3)
        wsem = (w0, w1, w2, w3)
        iot = _iota16()
        nvec = [iot + 16 * g for g in range(8)]
        jb0 = wid * _JPT
        tail_lo = _FULL * 128

        pltpu.sync_copy(idx_hbm.at[pl.ds(jb0 * 128, _JPT * 128)], idxv)
        pltpu.sync_copy(tail_hbm, tailv)

        def fixup(i, b):
            accs = []
            masks = []
            tidxs = []
            for g in range(8):
                iv = idxv[pl.ds(i * 128 + 16 * g, 16)]
                m = iv >= tail_lo
                masks.append(m)
                tidxs.append(jnp.maximum(iv - tail_lo, 0))
                accs.append(m.astype(jnp.int32))
            acc = accs[0]
            for a in accs[1:]:
                acc = acc + a
            nhit = lax.reduce_max(acc, axes=(0,))

            @pl.when(nhit > 0)
            def _():
                def dbody(d4, _):
                    for u in range(4):
                        d = d4 * 4 + u
                        dsp = _splat(d)
                        for g in range(8):
                            v = plsc.load_gather(
                                tailv, [tidxs[g], dsp], mask=masks[g])
                            plsc.store_scatter(
                                rows.at[b], [nvec[g], dsp], v,
                                mask=masks[g])
                    return _
                lax.fori_loop(0, 16, dbody, 0)

        def gather(i, b):
            return pltpu.make_async_copy(
                tab_hbm.at[idxv.at[pl.ds(i * 128, 128)]], rows.at[b], gsem[b])

        def gather_wait(b):
            pltpu.make_async_copy(
                tab_hbm.at[pl.ds(0, 128)], rows.at[b], gsem[b]).wait()

        def store(i, b):
            jb = jb0 + i
            s = jb // (_N // 128)
            tc = jb % (_N // 128)
            return pltpu.make_async_copy(
                trans.at[b], out_hbm.at[s, :, tc], wsem[b])

        def transpose(b):
            rows_b = rows.at[b]
            trans_b = trans.at[b]

            def kbody(k, _):
                dmod = (iot + k) & 15
                trrel = dmod >> 3
                rrel = dmod & 7
                for quar in range(4):
                    didx = dmod + 16 * quar
                    trv = trrel + 2 * quar
                    vs = []
                    for g in range(8):
                        vs.append(plsc.load_gather(rows_b, [nvec[g], didx]))
                    for g in range(8):
                        plsc.store_scatter(
                            trans_b, [trv, rrel, nvec[g]], vs[g])
                return _
            lax.fori_loop(0, 16, kbody, 0)

        for b in range(4):
            gather(b, b).start()
            store(b, b).start()

        def body(q, carry):
            for b in range(4):
                i = 4 * q + b
                gather_wait(b)
                fixup(i, b)
                store(i, b).wait()
                transpose(b)
                gather(jnp.minimum(i + 4, _JPT - 1), b).start()
                store(i, b).start()
            return carry

        lax.fori_loop(0, _JPT // 4, body, 0)

        for b in range(4):
            store(0, b).wait()
            gather_wait(b)

    return kb(table_lin, idx_j, tail64)


def kernel(inputs, table):
    table_t = table.T
    tab_lin = _detranspose(table_t).reshape(_V, _D)
    idx_j = inputs.T.reshape(_B).astype(jnp.int32)
    tail64 = table[_FULL * 128:]
    out5 = _gather_blocks(tab_lin, idx_j, tail64)
    return out5.transpose(2, 4, 0, 1, 3).reshape(_N, _S, _D)


# speedup vs baseline: 2.2956x; 1.0300x over previous
import functools

import jax
import jax.numpy as jnp
from jax import lax
from jax.experimental import pallas as pl
from jax.experimental.pallas import tpu as pltpu
from jax.experimental.pallas import tpu_sc as plsc

_D = 64
_NC, _NS = 2, 16
_NW = _NC * _NS
_V = 1000000
_FULL = _V // 128
_BPT = _FULL // _NW
_REM = _FULL - _BPT * _NW
_TAILN = _V - _FULL * 128

_N, _S = 4096, 200
_B = _N * _S
_JBLK = _B // 128
_JPT = _JBLK // _NW


def _mesh():
    return plsc.VectorSubcoreMesh(core_axis_name="c", subcore_axis_name="s")


def _iota16():
    return lax.iota(jnp.int32, 16)


def _splat(x):
    return jnp.full((16,), x, jnp.int32)


@jax.jit
def _detranspose(table_t):
    @functools.partial(
        pl.kernel,
        mesh=_mesh(),
        out_type=jax.ShapeDtypeStruct((_V * _D,), jnp.float32),
        scratch_types=[
            pltpu.VMEM((256, 128), jnp.float32),
            pltpu.VMEM((4 * 8192,), jnp.float32),
            pltpu.SemaphoreType.DMA,
            pltpu.SemaphoreType.DMA,
            pltpu.SemaphoreType.DMA,
            pltpu.SemaphoreType.DMA,
            pltpu.SemaphoreType.DMA,
            pltpu.SemaphoreType.DMA,
            pltpu.SemaphoreType.DMA,
            pltpu.SemaphoreType.DMA,
        ],
        compiler_params=pltpu.CompilerParams(use_tc_tiling_on_sc=True, needs_layout_passes=False, disable_bounds_checks=True),
    )
    def ka(tt_hbm, out_hbm, inb, outb,
           gi0, gi1, gi2, gi3, wo0, wo1, wo2, wo3):
        wid = lax.axis_index("s") * _NC + lax.axis_index("c")
        gsem = (gi0, gi1, gi2, gi3)
        wsem = (wo0, wo1, wo2, wo3)
        iot = _iota16()

        def blk_of(i):
            return i * _NW + wid

        def load(i, b):
            blk = blk_of(i)
            return [
                pltpu.make_async_copy(
                    tt_hbm.at[:, pl.ds(blk * 128, 128)],
                    inb.at[pl.ds(b * 64, 64)], gsem[b])
            ]

        def store(i, b):
            blk = blk_of(i)
            return pltpu.make_async_copy(
                outb.at[pl.ds(b * 8192, 8192)],
                out_hbm.at[pl.ds(blk * 8192, 8192)], wsem[b])

        def transpose(b):
            inb_b = inb.at[pl.ds(b * 64, 64)]
            outb_b = outb.at[pl.ds(b * 8192, 8192)]

            def kbody(k, _):
                nmod = (iot + k) & 15
                srel = (nmod << 6) + iot
                def load_q(quar):
                    d0 = 16 * quar
                    return [plsc.load_gather(
                        inb_b, [iot + d0, nmod + 16 * g]) for g in range(8)]

                def store_q(quar, vs):
                    d0 = 16 * quar
                    for g in range(8):
                        plsc.store_scatter(
                            outb_b, [srel + (16 * g * _D + d0)], vs[g])

                prev = load_q(0)
                for quar in range(1, 4):
                    d0 = 16 * quar
                    pd0 = d0 - 16
                    cur = []
                    for g in range(8):
                        cur.append(plsc.load_gather(
                            inb_b, [iot + d0, nmod + 16 * g]))
                        plsc.store_scatter(
                            outb_b, [srel + (16 * g * _D + pd0)], prev[g])
                    prev = cur
                store_q(3, prev)
                return _
            lax.fori_loop(0, 16, kbody, 0)

        for b in range(4):
            for c in load(b, b):
                c.start()
            store(b, b).start()

        def body(q, carry):
            for b in range(4):
                i = 4 * q + b
                for c in load(i, b):
                    c.wait()
                store(i, b).wait()
                transpose(b)
                for c in load(jnp.minimum(i + 4, _BPT - 1), b):
                    c.start()
                store(i, b).start()
            return carry

        lax.fori_loop(0, _BPT // 4, body, 0)

        for b in range(4):
            store(0, b).wait()
            for c in load(0, b):
                c.wait()

        @pl.when(wid < _REM)
        def _():
            blk = _FULL - _REM + wid
            pltpu.sync_copy(
                tt_hbm.at[:, pl.ds(blk * 128, 128)],
                inb.at[pl.ds(0, 64)])
            transpose(0)
            pltpu.sync_copy(outb.at[pl.ds(0, 8192)],
                            out_hbm.at[pl.ds(blk * 8192, 8192)])


    return ka(table_t)


@jax.jit
def _gather_blocks(table_lin, idx_j, tail64):
    @functools.partial(
        pl.kernel,
        mesh=_mesh(),
        out_type=jax.ShapeDtypeStruct((_S, 8, _N // 128, 8, 128), jnp.float32),
        scratch_types=[
            pltpu.VMEM((_JPT * 128,), jnp.int32),
            pltpu.VMEM((_TAILN, _D), jnp.float32),
            pltpu.VMEM((4, 128, _D), jnp.float32),
            pltpu.VMEM((4, 8, 8, 128), jnp.float32),
            pltpu.SemaphoreType.DMA,
            pltpu.SemaphoreType.DMA,
            pltpu.SemaphoreType.DMA,
            pltpu.SemaphoreType.DMA,
            pltpu.SemaphoreType.DMA,
            pltpu.SemaphoreType.DMA,
            pltpu.SemaphoreType.DMA,
            pltpu.SemaphoreType.DMA,
        ],
        compiler_params=pltpu.CompilerParams(use_tc_tiling_on_sc=False, needs_layout_passes=False, disable_bounds_checks=True),
    )
    def kb(tab_hbm, idx_hbm, tail_hbm, out_hbm, idxv, tailv, rows, trans,
           g0, g1, g2, g3, w0, w1, w2, w3):
        wid = lax.axis_index("s") * _NC + lax.axis_index("c")
        gsem = (g0, g1, g2, g3)
        wsem = (w0, w1, w2, w3)
        iot = _iota16()
        nvec = [iot + 16 * g for g in range(8)]
        jb0 = wid * _JPT
        tail_lo = _FULL * 128

        pltpu.sync_copy(idx_hbm.at[pl.ds(jb0 * 128, _JPT * 128)], idxv)
        pltpu.sync_copy(tail_hbm, tailv)

        def fixup(i, b):
            accs = []
            masks = []
            tidxs = []
            for g in range(8):
                iv = idxv[pl.ds(i * 128 + 16 * g, 16)]
                m = iv >= tail_lo
                masks.append(m)
                tidxs.append(jnp.maximum(iv - tail_lo, 0))
                accs.append(m.astype(jnp.int32))
            acc = accs[0]
            for a in accs[1:]:
                acc = acc + a
            nhit = lax.reduce_max(acc, axes=(0,))

            @pl.when(nhit > 0)
            def _():
                def dbody(d4, _):
                    for u in range(4):
                        d = d4 * 4 + u
                        dsp = _splat(d)
                        for g in range(8):
                            v = plsc.load_gather(
                                tailv, [tidxs[g], dsp], mask=masks[g])
                            plsc.store_scatter(
                                rows.at[b], [nvec[g], dsp], v,
                                mask=masks[g])
                    return _
                lax.fori_loop(0, 16, dbody, 0)

        def gather(i, b):
            return pltpu.make_async_copy(
                tab_hbm.at[idxv.at[pl.ds(i * 128, 128)]], rows.at[b], gsem[b])

        def gather_wait(b):
            pltpu.make_async_copy(
                tab_hbm.at[pl.ds(0, 128)], rows.at[b], gsem[b]).wait()

        def store(i, b):
            jb = jb0 + i
            s = jb // (_N // 128)
            tc = jb % (_N // 128)
            return pltpu.make_async_copy(
                trans.at[b], out_hbm.at[s, :, tc], wsem[b])

        def transpose(b):
            rows_b = rows.at[b]
            trans_b = trans.at[b]

            def kbody(k, _):
                dmod = (iot + k) & 15
                trrel = dmod >> 3
                rrel = dmod & 7
                prev = [plsc.load_gather(rows_b, [nvec[g], dmod])
                        for g in range(8)]
                for quar in range(1, 4):
                    didx = dmod + 16 * quar
                    ptrv = trrel + 2 * (quar - 1)
                    cur = []
                    for g in range(8):
                        cur.append(plsc.load_gather(rows_b, [nvec[g], didx]))
                        plsc.store_scatter(
                            trans_b, [ptrv, rrel, nvec[g]], prev[g])
                    prev = cur
                trv3 = trrel + 6
                for g in range(8):
                    plsc.store_scatter(trans_b, [trv3, rrel, nvec[g]], prev[g])
                return _
            lax.fori_loop(0, 16, kbody, 0)

        for b in range(4):
            gather(b, b).start()
            store(b, b).start()

        def body(q, carry):
            for b in range(4):
                i = 4 * q + b
                gather_wait(b)
                fixup(i, b)
                store(i, b).wait()
                transpose(b)
                gather(jnp.minimum(i + 4, _JPT - 1), b).start()
                store(i, b).start()
            return carry

        lax.fori_loop(0, _JPT // 4, body, 0)

        for b in range(4):
            store(0, b).wait()
            gather_wait(b)

    return kb(table_lin, idx_j, tail64)


def kernel(inputs, table):
    table_t = table.T
    tab_lin = _detranspose(table_t).reshape(_V, _D)
    idx_j = inputs.T.reshape(_B).astype(jnp.int32)
    tail64 = table[_FULL * 128:]
    out5 = _gather_blocks(tab_lin, idx_j, tail64)
    return out5.transpose(2, 4, 0, 1, 3).reshape(_N, _S, _D)
